# baseline (speedup 1.0000x reference)
import functools

import jax
import jax.numpy as jnp
from jax import lax
from jax.experimental import pallas as pl
from jax.experimental.pallas import tpu as pltpu
from jax.experimental.pallas import tpu_sc as plsc

_R_BINS = 32
_N = 4_000_000
_NC, _NS, _L = 2, 16, 16
_NW = _NC * _NS
_B = 128
_NBLK = _N // _B
_CBLK = 50
_CHUNK = _CBLK * _B
_PC = _CBLK * 3 * _B
_NCHUNKS = _NBLK // _CBLK
_CPW = (_NCHUNKS + _NW - 1) // _NW
_GROUPS = _CHUNK // _L
_NVAL = 7
_HIST = _NVAL * _R_BINS * _L
_ROWS = _NVAL * _R_BINS


def _sc_body(pos_hbm, vel_hbm, m_hbm, out_hbm,
             pos_v0, vel_v0, m_v0, pos_v1, vel_v1, m_v1,
             h0, h1, h2, h3, h4, h5, h6, acc_v, sem0, sem1):
    hists = (h0, h1, h2, h3, h4, h5, h6)
    cid = lax.axis_index("c")
    sid = lax.axis_index("s")
    wid = sid * _NC + cid

    lane = lax.iota(jnp.int32, _L)
    zero16 = jnp.zeros((_L,), jnp.float32)

    def _zero(j, carry):
        for hv in hists:
            hv[pl.ds(j * _L, _L)] = zero16
        return carry

    lax.fori_loop(0, _R_BINS * _L // _L, _zero, 0)

    bufs = ((pos_v0, vel_v0, m_v0, sem0), (pos_v1, vel_v1, m_v1, sem1))

    def _issue(c, buf):
        pv, vv, mv, sem = buf

        @pl.when(c < _NCHUNKS)
        def _():
            pltpu.async_copy(pos_hbm.at[pl.ds(c * _PC, _PC)], pv, sem)
            pltpu.async_copy(vel_hbm.at[pl.ds(c * _PC, _PC)], vv, sem)
            pltpu.async_copy(m_hbm.at[pl.ds(c * _CHUNK, _CHUNK)], mv, sem)

    def _process(c, buf):
        pv, vv, mv, sem = buf

        @pl.when(c < _NCHUNKS)
        def _():
            pltpu.make_async_copy(pos_hbm.at[pl.ds(0, _PC)], pv, sem).wait()
            pltpu.make_async_copy(vel_hbm.at[pl.ds(0, _PC)], vv, sem).wait()
            pltpu.make_async_copy(m_hbm.at[pl.ds(0, _CHUNK)], mv, sem).wait()

            @plsc.parallel_loop(0, _GROUPS, unroll=6)
            def group_body(g):
                po = (g >> 3) * (3 * _B) + (g & 7) * _L
                x = pv[pl.ds(po, _L)]
                y = pv[pl.ds(po + _B, _L)]
                vx = vv[pl.ds(po, _L)]
                vy = vv[pl.ds(po + _B, _L)]
                vz = vv[pl.ds(po + 2 * _B, _L)]
                m = mv[pl.ds(g * _L, _L)]

                s = x * x + y * y
                inv = plsc.bitcast(
                    jnp.int32(0x5F3759DF) - (plsc.bitcast(s, jnp.int32) >> 1),
                    jnp.float32)
                h = -0.5 * s
                inv = inv * (1.5 + h * inv * inv)
                inv = inv * (1.5 + h * inv * inv)

                u = (s * inv) * 8.0
                i0 = u.astype(jnp.int32)
                fi = i0.astype(jnp.float32)
                s64 = s * 64.0
                fh = fi + 1.0
                i1 = jnp.where(s64 >= fh * fh, i0 + 1, i0)
                i1 = jnp.where(s64 < fi * fi, i1 - 1, i1)
                w = jnp.where(i1 < _R_BINS, m, 0.0)
                ic = jnp.minimum(i1, _R_BINS - 1)

                nr = x * vx + y * vy
                nphi = y * vx - x * vy
                vr = nr * inv
                vphi = nphi * inv
                wvr = w * vr
                wvphi = w * vphi
                wvz = w * vz
                bidx = ic * _L + lane
                plsc.addupdate_scatter(h0, [bidx], w)
                plsc.addupdate_scatter(h1, [bidx], wvr)
                plsc.addupdate_scatter(h2, [bidx], wvr * vr)
                plsc.addupdate_scatter(h3, [bidx], wvphi)
                plsc.addupdate_scatter(h4, [bidx], wvphi * vphi)
                plsc.addupdate_scatter(h5, [bidx], wvz)
                plsc.addupdate_scatter(h6, [bidx], wvz * vz)

    _issue(wid, bufs[0])

    def chunk_pair(u, carry):
        t0 = u * 2
        c0 = wid + t0 * _NW
        c1 = c0 + _NW
        _issue(c1, bufs[1])
        _process(c0, bufs[0])
        _issue(c1 + _NW, bufs[0])
        _process(c1, bufs[1])
        return carry

    lax.fori_loop(0, _CPW // 2, chunk_pair, 0)

    lidx = lane * _L
    for k, hv in enumerate(hists):
        for j in range(_R_BINS // _L):
            b = j * (_L * _L)
            accv = zero16
            for l in range(_L):
                accv = accv + plsc.load_gather(hv, [lidx + (b + l)])
            acc_v[pl.ds(k * _R_BINS + j * _L, _L)] = accv

    pltpu.sync_copy(acc_v, out_hbm.at[wid])


_sc_hist = functools.partial(
    pl.kernel,
    out_type=jax.ShapeDtypeStruct((_NW, _ROWS), jnp.float32),
    mesh=plsc.VectorSubcoreMesh(
        core_axis_name="c", subcore_axis_name="s",
        num_cores=_NC, num_subcores=_NS),
    compiler_params=pltpu.CompilerParams(needs_layout_passes=False),
    scratch_types=[
        pltpu.VMEM((_PC,), jnp.float32),
        pltpu.VMEM((_PC,), jnp.float32),
        pltpu.VMEM((_CHUNK,), jnp.float32),
        pltpu.VMEM((_PC,), jnp.float32),
        pltpu.VMEM((_PC,), jnp.float32),
        pltpu.VMEM((_CHUNK,), jnp.float32),
        pltpu.VMEM((_R_BINS * _L,), jnp.float32),
        pltpu.VMEM((_R_BINS * _L,), jnp.float32),
        pltpu.VMEM((_R_BINS * _L,), jnp.float32),
        pltpu.VMEM((_R_BINS * _L,), jnp.float32),
        pltpu.VMEM((_R_BINS * _L,), jnp.float32),
        pltpu.VMEM((_R_BINS * _L,), jnp.float32),
        pltpu.VMEM((_R_BINS * _L,), jnp.float32),
        pltpu.VMEM((_ROWS,), jnp.float32),
        pltpu.SemaphoreType.DMA,
        pltpu.SemaphoreType.DMA,
    ],
)(_sc_body)


def _epi_body(p_ref, o_ref):
    s = jnp.sum(p_ref[:], axis=0)
    mass = s[0:1, :]
    vr = s[1:2] / mass
    vr2 = s[2:3] / mass
    vphi = s[3:4] / mass
    vphi2 = s[4:5] / mass
    vz = s[5:6] / mass
    vz2 = s[6:7] / mass
    o_ref[:] = jnp.concatenate([
        vphi, jnp.sqrt(vphi2 - vphi * vphi),
        vr, jnp.sqrt(vr2 - vr * vr),
        vz, jnp.sqrt(vz2 - vz * vz)], axis=0)


def _to_block_major(a):
    return a.reshape(_NBLK, _B, 3).transpose(0, 2, 1).reshape(-1)


def kernel(positions, velocities, masses):
    partials = _sc_hist(
        _to_block_major(positions), _to_block_major(velocities), masses)
    p3 = partials.reshape(_NW, _NVAL, _R_BINS)
    return pl.pallas_call(
        _epi_body,
        out_shape=jax.ShapeDtypeStruct((6, _R_BINS), jnp.float32),
    )(p3)


# speedup vs baseline: 1.0930x; 1.0930x over previous
import functools

import jax
import jax.numpy as jnp
from jax import lax
from jax.experimental import pallas as pl
from jax.experimental.pallas import tpu as pltpu
from jax.experimental.pallas import tpu_sc as plsc

_R_BINS = 32
_N = 4_000_000
_NC, _NS, _L = 2, 16, 16
_NW = _NC * _NS
_B = 128
_NBLK = _N // _B
_CBLK = 50
_CHUNK = _CBLK * _B
_PC = _CBLK * 3 * _B
_NCHUNKS = _NBLK // _CBLK
_CPW = (_NCHUNKS + _NW - 1) // _NW
_GROUPS = _CHUNK // _L
_NVAL = 7
_HIST = _NVAL * _R_BINS * _L
_ROWS = _NVAL * _R_BINS


def _sc_body(pos_hbm, vel_hbm, m_hbm, out_hbm,
             pos_v0, vel_v0, m_v0, pos_v1, vel_v1, m_v1,
             h0, h1, h2, h3, h4, h5, h6, acc_v, sem0, sem1):
    hists = (h0, h1, h2, h3, h4, h5, h6)
    cid = lax.axis_index("c")
    sid = lax.axis_index("s")
    wid = sid * _NC + cid

    lane = lax.iota(jnp.int32, _L)
    zero16 = jnp.zeros((_L,), jnp.float32)

    def _zero(j, carry):
        for hv in hists:
            hv[pl.ds(j * _L, _L)] = zero16
        return carry

    lax.fori_loop(0, _R_BINS * _L // _L, _zero, 0)

    bufs = ((pos_v0, vel_v0, m_v0, sem0), (pos_v1, vel_v1, m_v1, sem1))

    def _issue(c, buf):
        pv, vv, mv, sem = buf

        @pl.when(c < _NCHUNKS)
        def _():
            pltpu.async_copy(pos_hbm.at[pl.ds(c * _PC, _PC)], pv, sem)
            pltpu.async_copy(vel_hbm.at[pl.ds(c * _PC, _PC)], vv, sem)
            pltpu.async_copy(m_hbm.at[pl.ds(c * _CHUNK, _CHUNK)], mv, sem)

    def _process(c, buf):
        pv, vv, mv, sem = buf

        @pl.when(c < _NCHUNKS)
        def _():
            pltpu.make_async_copy(pos_hbm.at[pl.ds(0, _PC)], pv, sem).wait()
            pltpu.make_async_copy(vel_hbm.at[pl.ds(0, _PC)], vv, sem).wait()
            pltpu.make_async_copy(m_hbm.at[pl.ds(0, _CHUNK)], mv, sem).wait()

            @plsc.parallel_loop(0, _GROUPS, unroll=2)
            def group_body(g):
                po = (g >> 3) * (3 * _B) + (g & 7) * _L
                x = pv[pl.ds(po, _L)]
                y = pv[pl.ds(po + _B, _L)]
                vx = vv[pl.ds(po, _L)]
                vy = vv[pl.ds(po + _B, _L)]
                vz = vv[pl.ds(po + 2 * _B, _L)]
                m = mv[pl.ds(g * _L, _L)]

                s = x * x + y * y
                inv = plsc.bitcast(
                    jnp.int32(0x5F3759DF) - (plsc.bitcast(s, jnp.int32) >> 1),
                    jnp.float32)
                h = -0.5 * s
                inv = inv * (1.5 + h * inv * inv)
                inv = inv * (1.5 + h * inv * inv)

                u = (s * inv) * 8.0
                i0 = u.astype(jnp.int32)
                fi = i0.astype(jnp.float32)
                s64 = s * 64.0
                fh = fi + 1.0
                i1 = jnp.where(s64 >= fh * fh, i0 + 1, i0)
                i1 = jnp.where(s64 < fi * fi, i1 - 1, i1)
                w = jnp.where(i1 < _R_BINS, m, 0.0)
                ic = jnp.minimum(i1, _R_BINS - 1)

                nr = x * vx + y * vy
                nphi = y * vx - x * vy
                vr = nr * inv
                vphi = nphi * inv
                wvr = w * vr
                wvphi = w * vphi
                wvz = w * vz
                bidx = ic * _L + lane
                plsc.addupdate_scatter(h0, [bidx], w)
                plsc.addupdate_scatter(h1, [bidx], wvr)
                plsc.addupdate_scatter(h2, [bidx], wvr * vr)
                plsc.addupdate_scatter(h3, [bidx], wvphi)
                plsc.addupdate_scatter(h4, [bidx], wvphi * vphi)
                plsc.addupdate_scatter(h5, [bidx], wvz)
                plsc.addupdate_scatter(h6, [bidx], wvz * vz)

    _issue(wid, bufs[0])

    def chunk_pair(u, carry):
        t0 = u * 2
        c0 = wid + t0 * _NW
        c1 = c0 + _NW
        _issue(c1, bufs[1])
        _process(c0, bufs[0])
        _issue(c1 + _NW, bufs[0])
        _process(c1, bufs[1])
        return carry

    lax.fori_loop(0, _CPW // 2, chunk_pair, 0)

    lidx = lane * _L
    for k, hv in enumerate(hists):
        for j in range(_R_BINS // _L):
            b = j * (_L * _L)
            accv = zero16
            for l in range(_L):
                accv = accv + plsc.load_gather(hv, [lidx + (b + l)])
            acc_v[pl.ds(k * _R_BINS + j * _L, _L)] = accv

    pltpu.sync_copy(acc_v, out_hbm.at[wid])


_sc_hist = functools.partial(
    pl.kernel,
    out_type=jax.ShapeDtypeStruct((_NW, _ROWS), jnp.float32),
    mesh=plsc.VectorSubcoreMesh(
        core_axis_name="c", subcore_axis_name="s",
        num_cores=_NC, num_subcores=_NS),
    compiler_params=pltpu.CompilerParams(needs_layout_passes=False),
    scratch_types=[
        pltpu.VMEM((_PC,), jnp.float32),
        pltpu.VMEM((_PC,), jnp.float32),
        pltpu.VMEM((_CHUNK,), jnp.float32),
        pltpu.VMEM((_PC,), jnp.float32),
        pltpu.VMEM((_PC,), jnp.float32),
        pltpu.VMEM((_CHUNK,), jnp.float32),
        pltpu.VMEM((_R_BINS * _L,), jnp.float32),
        pltpu.VMEM((_R_BINS * _L,), jnp.float32),
        pltpu.VMEM((_R_BINS * _L,), jnp.float32),
        pltpu.VMEM((_R_BINS * _L,), jnp.float32),
        pltpu.VMEM((_R_BINS * _L,), jnp.float32),
        pltpu.VMEM((_R_BINS * _L,), jnp.float32),
        pltpu.VMEM((_R_BINS * _L,), jnp.float32),
        pltpu.VMEM((_ROWS,), jnp.float32),
        pltpu.SemaphoreType.DMA,
        pltpu.SemaphoreType.DMA,
    ],
)(_sc_body)


def _epi_body(p_ref, o_ref):
    s = jnp.sum(p_ref[:], axis=0)
    mass = s[0:1, :]
    vr = s[1:2] / mass
    vr2 = s[2:3] / mass
    vphi = s[3:4] / mass
    vphi2 = s[4:5] / mass
    vz = s[5:6] / mass
    vz2 = s[6:7] / mass
    o_ref[:] = jnp.concatenate([
        vphi, jnp.sqrt(vphi2 - vphi * vphi),
        vr, jnp.sqrt(vr2 - vr * vr),
        vz, jnp.sqrt(vz2 - vz * vz)], axis=0)


def _to_block_major(a):
    return a.reshape(_NBLK, _B, 3).transpose(0, 2, 1).reshape(-1)


def kernel(positions, velocities, masses):
    partials = _sc_hist(
        _to_block_major(positions), _to_block_major(velocities), masses)
    p3 = partials.reshape(_NW, _NVAL, _R_BINS)
    return pl.pallas_call(
        _epi_body,
        out_shape=jax.ShapeDtypeStruct((6, _R_BINS), jnp.float32),
    )(p3)


# speedup vs baseline: 1.1166x; 1.0216x over previous
import functools

import jax
import jax.numpy as jnp
from jax import lax
from jax.experimental import pallas as pl
from jax.experimental.pallas import tpu as pltpu
from jax.experimental.pallas import tpu_sc as plsc

_R_BINS = 32
_N = 4_000_000
_NC, _NS, _L = 2, 16, 16
_NW = _NC * _NS
_B = 128
_NBLK = _N // _B
_CBLK = 50
_CHUNK = _CBLK * _B
_PC = _CBLK * 3 * _B
_NCHUNKS = _NBLK // _CBLK
_CPW = (_NCHUNKS + _NW - 1) // _NW
_GROUPS = _CHUNK // _L
_NVAL = 7
_HIST = _NVAL * _R_BINS * _L
_ROWS = _NVAL * _R_BINS


def _sc_body(pos_hbm, vel_hbm, m_hbm, out_hbm,
             pos_v0, vel_v0, m_v0, pos_v1, vel_v1, m_v1,
             h0, h1, h2, h3, h4, h5, h6, acc_v, sem0, sem1):
    hists = (h0, h1, h2, h3, h4, h5, h6)
    cid = lax.axis_index("c")
    sid = lax.axis_index("s")
    wid = sid * _NC + cid

    lane = lax.iota(jnp.int32, _L)
    zero16 = jnp.zeros((_L,), jnp.float32)

    def _zero(j, carry):
        for hv in hists:
            hv[pl.ds(j * _L, _L)] = zero16
        return carry

    lax.fori_loop(0, _R_BINS * _L // _L, _zero, 0)

    bufs = ((pos_v0, vel_v0, m_v0, sem0), (pos_v1, vel_v1, m_v1, sem1))

    def _issue(c, buf):
        pv, vv, mv, sem = buf

        @pl.when(c < _NCHUNKS)
        def _():
            pltpu.async_copy(pos_hbm.at[pl.ds(c * _PC, _PC)], pv, sem)
            pltpu.async_copy(vel_hbm.at[pl.ds(c * _PC, _PC)], vv, sem)
            pltpu.async_copy(m_hbm.at[pl.ds(c * _CHUNK, _CHUNK)], mv, sem)

    def _process(c, buf):
        pv, vv, mv, sem = buf

        @pl.when(c < _NCHUNKS)
        def _():
            pltpu.make_async_copy(pos_hbm.at[pl.ds(0, _PC)], pv, sem).wait()
            pltpu.make_async_copy(vel_hbm.at[pl.ds(0, _PC)], vv, sem).wait()
            pltpu.make_async_copy(m_hbm.at[pl.ds(0, _CHUNK)], mv, sem).wait()

            @plsc.parallel_loop(0, _GROUPS, unroll=4)
            def group_body(g):
                po = (g >> 3) * (3 * _B) + (g & 7) * _L
                x = pv[pl.ds(po, _L)]
                y = pv[pl.ds(po + _B, _L)]
                vx = vv[pl.ds(po, _L)]
                vy = vv[pl.ds(po + _B, _L)]
                vz = vv[pl.ds(po + 2 * _B, _L)]
                m = mv[pl.ds(g * _L, _L)]

                s = x * x + y * y
                inv = plsc.bitcast(
                    jnp.int32(0x5F3759DF) - (plsc.bitcast(s, jnp.int32) >> 1),
                    jnp.float32)
                h = -0.5 * s
                inv = inv * (1.5 + h * inv * inv)
                inv = inv * (1.5 + h * inv * inv)

                u = (s * inv) * 8.00016
                i0 = u.astype(jnp.int32)
                fi = i0.astype(jnp.float32)
                s64 = s * 64.0
                i1 = jnp.where(s64 < fi * fi, i0 - 1, i0)
                w = jnp.where(i1 < _R_BINS, m, 0.0)
                ic = jnp.minimum(i1, _R_BINS - 1)

                nr = x * vx + y * vy
                nphi = y * vx - x * vy
                vr = nr * inv
                vphi = nphi * inv
                wvr = w * vr
                wvphi = w * vphi
                wvz = w * vz
                bidx = ic * _L + lane
                plsc.addupdate_scatter(h0, [bidx], w)
                plsc.addupdate_scatter(h1, [bidx], wvr)
                plsc.addupdate_scatter(h2, [bidx], wvr * vr)
                plsc.addupdate_scatter(h3, [bidx], wvphi)
                plsc.addupdate_scatter(h4, [bidx], wvphi * vphi)
                plsc.addupdate_scatter(h5, [bidx], wvz)
                plsc.addupdate_scatter(h6, [bidx], wvz * vz)

    _issue(wid, bufs[0])

    def chunk_pair(u, carry):
        t0 = u * 2
        c0 = wid + t0 * _NW
        c1 = c0 + _NW
        _issue(c1, bufs[1])
        _process(c0, bufs[0])
        _issue(c1 + _NW, bufs[0])
        _process(c1, bufs[1])
        return carry

    lax.fori_loop(0, _CPW // 2, chunk_pair, 0)

    lidx = lane * _L
    for k, hv in enumerate(hists):
        for j in range(_R_BINS // _L):
            b = j * (_L * _L)
            accv = zero16
            for l in range(_L):
                accv = accv + plsc.load_gather(hv, [lidx + (b + l)])
            acc_v[pl.ds(k * _R_BINS + j * _L, _L)] = accv

    pltpu.sync_copy(acc_v, out_hbm.at[wid])


_sc_hist = functools.partial(
    pl.kernel,
    out_type=jax.ShapeDtypeStruct((_NW, _ROWS), jnp.float32),
    mesh=plsc.VectorSubcoreMesh(
        core_axis_name="c", subcore_axis_name="s",
        num_cores=_NC, num_subcores=_NS),
    compiler_params=pltpu.CompilerParams(needs_layout_passes=False),
    scratch_types=[
        pltpu.VMEM((_PC,), jnp.float32),
        pltpu.VMEM((_PC,), jnp.float32),
        pltpu.VMEM((_CHUNK,), jnp.float32),
        pltpu.VMEM((_PC,), jnp.float32),
        pltpu.VMEM((_PC,), jnp.float32),
        pltpu.VMEM((_CHUNK,), jnp.float32),
        pltpu.VMEM((_R_BINS * _L,), jnp.float32),
        pltpu.VMEM((_R_BINS * _L,), jnp.float32),
        pltpu.VMEM((_R_BINS * _L,), jnp.float32),
        pltpu.VMEM((_R_BINS * _L,), jnp.float32),
        pltpu.VMEM((_R_BINS * _L,), jnp.float32),
        pltpu.VMEM((_R_BINS * _L,), jnp.float32),
        pltpu.VMEM((_R_BINS * _L,), jnp.float32),
        pltpu.VMEM((_ROWS,), jnp.float32),
        pltpu.SemaphoreType.DMA,
        pltpu.SemaphoreType.DMA,
    ],
)(_sc_body)


def _epi_body(p_ref, o_ref):
    s = jnp.sum(p_ref[:], axis=0)
    mass = s[0:1, :]
    vr = s[1:2] / mass
    vr2 = s[2:3] / mass
    vphi = s[3:4] / mass
    vphi2 = s[4:5] / mass
    vz = s[5:6] / mass
    vz2 = s[6:7] / mass
    o_ref[:] = jnp.concatenate([
        vphi, jnp.sqrt(vphi2 - vphi * vphi),
        vr, jnp.sqrt(vr2 - vr * vr),
        vz, jnp.sqrt(vz2 - vz * vz)], axis=0)


def _to_block_major(a):
    return a.reshape(_NBLK, _B, 3).transpose(0, 2, 1).reshape(-1)


def kernel(positions, velocities, masses):
    partials = _sc_hist(
        _to_block_major(positions), _to_block_major(velocities), masses)
    p3 = partials.reshape(_NW, _NVAL, _R_BINS)
    return pl.pallas_call(
        _epi_body,
        out_shape=jax.ShapeDtypeStruct((6, _R_BINS), jnp.float32),
    )(p3)
